# Initial kernel scaffold; baseline (speedup 1.0000x reference)
import jax
import jax.numpy as jnp
from jax.experimental import pallas as pl


def kernel(inputs):
    raise NotImplementedError("write your pallas kernel here")


# speedup vs baseline: 4.6825x; 4.6825x over previous
import functools
import itertools

import jax
import jax.numpy as jnp
import numpy as np
from jax.experimental import pallas as pl
from jax.experimental.pallas import tpu as pltpu

_N_BITS = 20
_ORDER = 4
_K_PAD = 32


def _membership_matrix():
    combos = []
    for size in range(1, _ORDER + 1):
        combos.extend(itertools.combinations(range(_N_BITS), size))
    num_terms = len(combos)
    a = np.zeros((_K_PAD, num_terms), dtype=np.float32)
    for t, combo in enumerate(combos):
        for i in combo:
            a[i, t] = 1.0
    return a, num_terms


_A_NP, _NUM_TERMS = _membership_matrix()

_B_BLK = 1024
_T_BLK = 512
_LOG_CLAMP = -80.0


def _parity_kernel(x_ref, a_ref, o_ref):
    x = x_ref[...]
    a = a_ref[...]
    logmag = jnp.maximum(jnp.log(jnp.abs(x)), _LOG_CLAMP)
    negbit = jnp.where(x < 0, 1.0, 0.0).astype(jnp.float32)
    logp = jax.lax.dot(logmag, a, precision=jax.lax.Precision.HIGHEST,
                       preferred_element_type=jnp.float32)
    negc = jax.lax.dot(negbit, a, precision=jax.lax.Precision.HIGHEST,
                       preferred_element_type=jnp.float32)
    par = negc - 2.0 * jnp.floor(negc * 0.5)
    sign = 1.0 - 2.0 * par
    o_ref[...] = sign * jnp.exp(logp)


@jax.jit
def kernel(inputs):
    batch = inputs.shape[0]
    x = jnp.pad(inputs, ((0, 0), (0, _K_PAD - _N_BITS)))
    a = jnp.asarray(_A_NP)
    grid = (batch // _B_BLK, pl.cdiv(_NUM_TERMS, _T_BLK))
    return pl.pallas_call(
        _parity_kernel,
        grid=grid,
        in_specs=[
            pl.BlockSpec((_B_BLK, _K_PAD), lambda i, j: (i, 0)),
            pl.BlockSpec((_K_PAD, _T_BLK), lambda i, j: (0, j)),
        ],
        out_specs=pl.BlockSpec((_B_BLK, _T_BLK), lambda i, j: (i, j)),
        out_shape=jax.ShapeDtypeStruct((batch, _NUM_TERMS), jnp.float32),
    )(x, a)


# speedup vs baseline: 8.1230x; 1.7348x over previous
import functools
import itertools

import jax
import jax.numpy as jnp
import numpy as np
from jax.experimental import pallas as pl
from jax.experimental.pallas import tpu as pltpu

_N_BITS = 20
_ORDER = 4
_K_PAD = 32


def _membership_matrix():
    combos = []
    for size in range(1, _ORDER + 1):
        combos.extend(itertools.combinations(range(_N_BITS), size))
    num_terms = len(combos)
    a = np.zeros((_K_PAD, num_terms), dtype=np.float32)
    for t, combo in enumerate(combos):
        for i in combo:
            a[i, t] = 1.0
    return a, num_terms


_A_NP, _NUM_TERMS = _membership_matrix()

_B_BLK = 1024
_T_BLK = 512
_LOG_CLAMP = -80.0


def _parity_kernel(x_ref, a_ref, o_ref):
    x = x_ref[...]
    a = a_ref[...]
    logmag = jnp.maximum(jnp.log(jnp.abs(x)), _LOG_CLAMP)
    negbit = jnp.where(x < 0, 1.0, 0.0).astype(jnp.float32)
    a_bf = a.astype(jnp.bfloat16)
    hi = logmag.astype(jnp.bfloat16)
    lo = (logmag - hi.astype(jnp.float32)).astype(jnp.bfloat16)
    logp = (jax.lax.dot(hi, a_bf, preferred_element_type=jnp.float32)
            + jax.lax.dot(lo, a_bf, preferred_element_type=jnp.float32))
    negc = jax.lax.dot(negbit.astype(jnp.bfloat16), a_bf,
                       preferred_element_type=jnp.float32)
    par = negc - 2.0 * jnp.floor(negc * 0.5)
    sign = 1.0 - 2.0 * par
    o_ref[...] = sign * jnp.exp(logp)


@jax.jit
def kernel(inputs):
    batch = inputs.shape[0]
    x = jnp.pad(inputs, ((0, 0), (0, _K_PAD - _N_BITS)))
    a = jnp.asarray(_A_NP)
    grid = (batch // _B_BLK, pl.cdiv(_NUM_TERMS, _T_BLK))
    return pl.pallas_call(
        _parity_kernel,
        grid=grid,
        in_specs=[
            pl.BlockSpec((_B_BLK, _K_PAD), lambda i, j: (i, 0)),
            pl.BlockSpec((_K_PAD, _T_BLK), lambda i, j: (0, j)),
        ],
        out_specs=pl.BlockSpec((_B_BLK, _T_BLK), lambda i, j: (i, j)),
        out_shape=jax.ShapeDtypeStruct((batch, _NUM_TERMS), jnp.float32),
    )(x, a)
